# baseline (speedup 1.0000x reference)
import jax
import jax.numpy as jnp
from jax.experimental import pallas as pl
from jax.experimental.pallas import tpu as pltpu

T = 2048
D_IN = 768
D_H = 1536
D_OUT = 768
E = 8
K = 3

HK = 4
BH = D_H // HK


def _routing_kernel(x_ref, wg_ref, bg_ref, w_ref):
    scores = jnp.dot(x_ref[...].astype(jnp.bfloat16),
                     wg_ref[...].astype(jnp.bfloat16),
                     preferred_element_type=jnp.float32) + bg_ref[...]
    lane = jax.lax.broadcasted_iota(jnp.int32, scores.shape, 1)
    neg = jnp.float32(-3.4e38)
    s = scores
    mask = jnp.zeros(scores.shape, dtype=jnp.bool_)
    for _ in range(K):
        m = jnp.max(s, axis=1, keepdims=True)
        is_max = s == m
        first = jnp.min(jnp.where(is_max, lane, E), axis=1, keepdims=True)
        sel = lane == first
        mask = jnp.logical_or(mask, sel)
        s = jnp.where(sel, neg, s)
    p = jax.nn.softmax(scores, axis=1)
    w = jnp.where(mask, p, 0.0)
    w = w / (jnp.sum(w, axis=1, keepdims=True) + 1e-8)
    w_ref[...] = w


def _ffn_kernel(w_ref, x_ref, w1_ref, b1_ref, w2_ref, b2_ref, out_ref, xb):
    e = pl.program_id(0)
    hk = pl.program_id(1)

    @pl.when(jnp.logical_and(e == 0, hk == 0))
    def _():
        xb[...] = x_ref[...].astype(jnp.bfloat16)

    h = jnp.dot(xb[...], w1_ref[0].astype(jnp.bfloat16),
                preferred_element_type=jnp.float32)
    h = jnp.maximum(h + b1_ref[0], 0.0)
    part = jnp.dot(h.astype(jnp.bfloat16), w2_ref[0].astype(jnp.bfloat16),
                   preferred_element_type=jnp.float32)

    wblk = w_ref[...]
    lane = jax.lax.broadcasted_iota(jnp.int32, wblk.shape, 1)
    wcol = jnp.sum(jnp.where(lane == e, wblk, 0.0), axis=1, keepdims=True)

    @pl.when(hk == 0)
    def _():
        contrib = (part + b2_ref[0]) * wcol

        @pl.when(e == 0)
        def _():
            out_ref[...] = contrib

        @pl.when(e > 0)
        def _():
            out_ref[...] = out_ref[...] + contrib

    @pl.when(hk > 0)
    def _():
        out_ref[...] = out_ref[...] + part * wcol


def kernel(x, Wg, bg, W1, b1, W2, b2):
    bg2 = bg.reshape(1, E)
    weights = pl.pallas_call(
        _routing_kernel,
        out_shape=jax.ShapeDtypeStruct((T, E), jnp.float32),
        in_specs=[
            pl.BlockSpec((T, D_IN), lambda: (0, 0)),
            pl.BlockSpec((D_IN, E), lambda: (0, 0)),
            pl.BlockSpec((1, E), lambda: (0, 0)),
        ],
        out_specs=pl.BlockSpec((T, E), lambda: (0, 0)),
    )(x, Wg, bg2)

    out = pl.pallas_call(
        _ffn_kernel,
        grid=(E, HK),
        out_shape=jax.ShapeDtypeStruct((T, D_OUT), jnp.float32),
        in_specs=[
            pl.BlockSpec((T, E), lambda e, h: (0, 0)),
            pl.BlockSpec((T, D_IN), lambda e, h: (0, 0)),
            pl.BlockSpec((1, D_IN, BH), lambda e, h: (e, 0, h)),
            pl.BlockSpec((1, 1, BH), lambda e, h: (e, 0, h)),
            pl.BlockSpec((1, BH, D_OUT), lambda e, h: (e, h, 0)),
            pl.BlockSpec((1, 1, D_OUT), lambda e, h: (e, 0, 0)),
        ],
        out_specs=pl.BlockSpec((T, D_OUT), lambda e, h: (0, 0)),
        scratch_shapes=[
            pltpu.VMEM((T, D_IN), jnp.bfloat16),
        ],
        compiler_params=pltpu.CompilerParams(
            dimension_semantics=("arbitrary", "arbitrary"),
        ),
    )(weights, x, W1, b1.reshape(E, 1, D_H), W2, b2.reshape(E, 1, D_OUT))
    return out


# speedup vs baseline: 1.2619x; 1.2619x over previous
import jax
import jax.numpy as jnp
from jax.experimental import pallas as pl
from jax.experimental.pallas import tpu as pltpu

T = 2048
D_IN = 768
D_H = 1536
D_OUT = 768
E = 8
K = 3

BT = 256
NT = T // BT


def _routing_kernel(x_ref, wg_ref, bg_ref, w_ref):
    scores = jnp.dot(x_ref[...].astype(jnp.bfloat16),
                     wg_ref[...].astype(jnp.bfloat16),
                     preferred_element_type=jnp.float32) + bg_ref[...]
    lane = jax.lax.broadcasted_iota(jnp.int32, scores.shape, 1)
    neg = jnp.float32(-3.4e38)
    s = scores
    mask = jnp.zeros(scores.shape, dtype=jnp.bool_)
    for _ in range(K):
        m = jnp.max(s, axis=1, keepdims=True)
        is_max = s == m
        first = jnp.min(jnp.where(is_max, lane, E), axis=1, keepdims=True)
        sel = lane == first
        mask = jnp.logical_or(mask, sel)
        s = jnp.where(sel, neg, s)
    p = jax.nn.softmax(scores, axis=1)
    w = jnp.where(mask, p, 0.0)
    w = w / (jnp.sum(w, axis=1, keepdims=True) + 1e-8)
    w_ref[...] = w


def _w_copy(w1_hbm, w2_hbm, w1r, w2r, sem1, sem2, expert, slot):
    c1 = pltpu.make_async_copy(w1_hbm.at[expert], w1r.at[slot], sem1.at[slot])
    c2 = pltpu.make_async_copy(w2_hbm.at[expert], w2r.at[slot], sem2.at[slot])
    return c1, c2


def _ffn_kernel(w_ref, x_ref, w1_hbm, b1_ref, w2_hbm, b2_ref, out_ref,
                w1r, w2r, xb, sem1, sem2):
    e = pl.program_id(0)
    t = pl.program_id(1)
    slot = jax.lax.rem(e, 2)

    @pl.when(t == 0)
    def _():
        @pl.when(e == 0)
        def _():
            ca1, ca2 = _w_copy(w1_hbm, w2_hbm, w1r, w2r, sem1, sem2, 0, 0)
            cb1, cb2 = _w_copy(w1_hbm, w2_hbm, w1r, w2r, sem1, sem2, 1, 1)
            ca1.start()
            ca2.start()
            cb1.start()
            cb2.start()

        @pl.when(jnp.logical_and(e >= 1, e <= E - 2))
        def _():
            nxt = e + 1
            c1, c2 = _w_copy(w1_hbm, w2_hbm, w1r, w2r, sem1, sem2,
                             nxt, jax.lax.rem(nxt, 2))
            c1.start()
            c2.start()

        c1, c2 = _w_copy(w1_hbm, w2_hbm, w1r, w2r, sem1, sem2, e, slot)
        c1.wait()
        c2.wait()

    rows = pl.ds(t * BT, BT)

    @pl.when(e == 0)
    def _():
        xb[rows, :] = x_ref[rows, :].astype(jnp.bfloat16)

    h = jnp.dot(xb[rows, :], w1r[slot].astype(jnp.bfloat16),
                preferred_element_type=jnp.float32)
    h = jnp.maximum(h + b1_ref[0], 0.0)
    o = jnp.dot(h.astype(jnp.bfloat16), w2r[slot].astype(jnp.bfloat16),
                preferred_element_type=jnp.float32)
    o = o + b2_ref[0]
    wblk = w_ref[rows, :]
    lane = jax.lax.broadcasted_iota(jnp.int32, wblk.shape, 1)
    wcol = jnp.sum(jnp.where(lane == e, wblk, 0.0), axis=1, keepdims=True)
    contrib = o * wcol

    @pl.when(e == 0)
    def _():
        out_ref[rows, :] = contrib

    @pl.when(e > 0)
    def _():
        out_ref[rows, :] = out_ref[rows, :] + contrib


def kernel(x, Wg, bg, W1, b1, W2, b2):
    bg2 = bg.reshape(1, E)
    weights = pl.pallas_call(
        _routing_kernel,
        out_shape=jax.ShapeDtypeStruct((T, E), jnp.float32),
        in_specs=[
            pl.BlockSpec((T, D_IN), lambda: (0, 0)),
            pl.BlockSpec((D_IN, E), lambda: (0, 0)),
            pl.BlockSpec((1, E), lambda: (0, 0)),
        ],
        out_specs=pl.BlockSpec((T, E), lambda: (0, 0)),
    )(x, Wg, bg2)

    out = pl.pallas_call(
        _ffn_kernel,
        grid=(E, NT),
        out_shape=jax.ShapeDtypeStruct((T, D_OUT), jnp.float32),
        in_specs=[
            pl.BlockSpec((T, E), lambda e, t: (0, 0)),
            pl.BlockSpec((T, D_IN), lambda e, t: (0, 0)),
            pl.BlockSpec(memory_space=pl.ANY),
            pl.BlockSpec((1, 1, D_H), lambda e, t: (e, 0, 0)),
            pl.BlockSpec(memory_space=pl.ANY),
            pl.BlockSpec((1, 1, D_OUT), lambda e, t: (e, 0, 0)),
        ],
        out_specs=pl.BlockSpec((T, D_OUT), lambda e, t: (0, 0)),
        scratch_shapes=[
            pltpu.VMEM((2, D_IN, D_H), jnp.float32),
            pltpu.VMEM((2, D_H, D_OUT), jnp.float32),
            pltpu.VMEM((T, D_IN), jnp.bfloat16),
            pltpu.SemaphoreType.DMA((2,)),
            pltpu.SemaphoreType.DMA((2,)),
        ],
        compiler_params=pltpu.CompilerParams(
            dimension_semantics=("arbitrary", "arbitrary"),
        ),
    )(weights, x, W1, b1.reshape(E, 1, D_H), W2, b2.reshape(E, 1, D_OUT))
    return out
